# Initial kernel scaffold; baseline (speedup 1.0000x reference)
import jax
import jax.numpy as jnp
from jax.experimental import pallas as pl


def kernel(x, edge_index, edge_attr, eW, eb, mW, mb, g0, b0, ge, be):
    raise NotImplementedError("write your pallas kernel here")


# speedup vs baseline: 3.8030x; 3.8030x over previous
import functools

import jax
import jax.numpy as jnp
from jax import lax
from jax.experimental import pallas as pl
from jax.experimental.pallas import tpu as pltpu
from jax.experimental.pallas import tpu_sc as plsc

_NC = 2
_NS = 16
_L = 16
_CH = 128


def _perm16(v, idx):
    dnums = lax.GatherDimensionNumbers(
        offset_dims=(), collapsed_slice_dims=(0,), start_index_map=(0,))
    return lax.gather(v, idx[:, None], dnums, (1,),
                      mode=lax.GatherScatterMode.PROMISE_IN_BOUNDS)


def _bsum16(v):
    idx = lax.iota(jnp.int32, _L)
    for k in (1, 2, 4, 8):
        v = v + _perm16(v, idx ^ k)
    return v


def _rsqrt16(v):
    i = lax.bitcast_convert_type(v, jnp.int32)
    y = lax.bitcast_convert_type(jnp.int32(0x5F3759DF) - (i >> 1),
                                 jnp.float32)
    for _ in range(3):
        y = y * (1.5 - 0.5 * v * y * y)
    return y


def _row_partition(N):
    rows = (N // _NS) & ~7
    rem = N - rows * _NS
    return rows, rem


def _edge_plan(E):
    NW = _NC * _NS
    ept = E // NW
    nfull = ept // _CH
    tail = ept - nfull * _CH
    return ept, nfull, tail


def _sc_edge_call(N, D, DE, E, src, dst, P, Q, C, ge, be):
    EPT, NFULL, TAIL = _edge_plan(E)
    ROWS, REM = _row_partition(N)

    mesh = plsc.VectorSubcoreMesh(core_axis_name="c", subcore_axis_name="s")

    scratch = [
        pltpu.VMEM((_CH,), jnp.int32),
        pltpu.VMEM((_CH,), jnp.int32),
        pltpu.VMEM((_CH, DE), jnp.float32),
        pltpu.VMEM((_CH, DE), jnp.float32),
        pltpu.VMEM((_CH, DE), jnp.float32),
        pltpu.VMEM((_CH, DE), jnp.float32),
        pltpu.VMEM((_CH, DE), jnp.float32),
        pltpu.VMEM((DE,), jnp.float32),
        pltpu.VMEM((DE,), jnp.float32),
        pltpu.VMEM_SHARED((N, DE), jnp.float32),
        pltpu.VMEM_SHARED((N, DE), jnp.float32),
        pltpu.SemaphoreType.DMA,
        pltpu.SemaphoreType.DMA,
        pltpu.SemaphoreType.DMA,
    ]
    if TAIL:
        scratch += [pltpu.VMEM((TAIL,), jnp.int32),
                    pltpu.VMEM((TAIL,), jnp.int32)]

    out_type = [
        jax.ShapeDtypeStruct((E, DE), jnp.float32),
        jax.ShapeDtypeStruct((_NC, N, DE), jnp.float32),
        jax.ShapeDtypeStruct((_NC, N, DE), jnp.float32),
    ]

    @functools.partial(
        pl.kernel, out_type=out_type, mesh=mesh, scratch_types=scratch,
        compiler_params=pltpu.CompilerParams(use_tc_tiling_on_sc=False))
    def body(src_hbm, dst_hbm, p_hbm, q_hbm, c_hbm, ge_hbm, be_hbm,
             eo_hbm, segE_hbm, cnt_hbm,
             srcv, dstv, bufP, bufQ, bufC, bufEO, onesv, gev, bev,
             sE, sC, s0, s1, s2, *tail_scratch):
        cid = lax.axis_index("c")
        sid = lax.axis_index("s")
        wid = sid * _NC + cid
        ebase0 = wid * EPT

        z16 = jnp.zeros((_L,), jnp.float32)
        one16 = jnp.full((_L,), 1.0, jnp.float32)

        def _zrow(i, carry):
            bufEO[i, :] = z16
            onesv[i, :] = one16
            return carry

        lax.fori_loop(0, _CH, _zrow, 0)
        pltpu.sync_copy(ge_hbm, gev)
        pltpu.sync_copy(be_hbm, bev)

        def _init_rows(base_r, nrows):
            off = 0
            while off < nrows:
                sz = min(_CH, nrows - off)
                zsrc = bufEO if sz == _CH else bufEO.at[pl.ds(0, sz)]
                pltpu.sync_copy(zsrc, sE.at[pl.ds(base_r + off, sz)])
                pltpu.sync_copy(zsrc, sC.at[pl.ds(base_r + off, sz)])
                off += sz

        _init_rows(sid * ROWS, ROWS)
        if REM:
            @pl.when(sid == _NS - 1)
            def _():
                _init_rows(_NS * ROWS, REM)
        plsc.subcore_barrier()

        def _sl(ref, ch):
            return ref if ch == _CH else ref.at[pl.ds(0, ch)]

        def _chunk(ebase, ch, srci, dsti):
            pltpu.sync_copy(src_hbm.at[pl.ds(ebase, ch)], srci)
            pltpu.sync_copy(dst_hbm.at[pl.ds(ebase, ch)], dsti)
            dP = pltpu.async_copy(p_hbm.at[srci], _sl(bufP, ch), s0)
            dQ = pltpu.async_copy(q_hbm.at[dsti], _sl(bufQ, ch), s1)
            dC = pltpu.async_copy(c_hbm.at[pl.ds(ebase, ch)],
                                  _sl(bufC, ch), s2)
            dP.wait(); dQ.wait(); dC.wait()

            gv = gev[:]
            bv = bev[:]

            def _edge(e):
                t = bufC[e, :] + bufP[e, :] + bufQ[e, :]
                mu = _bsum16(t) * (1.0 / _L)
                d = t - mu
                var = _bsum16(d * d) * (1.0 / _L)
                y = _rsqrt16(var + 1e-5)
                bufEO[e, :] = d * y * gv + bv

            U = 4
            def _ebody(i, carry):
                for u in range(U):
                    _edge(i * U + u)
                return carry

            lax.fori_loop(0, ch // U, _ebody, 0)

            pltpu.sync_copy(_sl(bufEO, ch), eo_hbm.at[pl.ds(ebase, ch)])
            pltpu.sync_copy(_sl(bufEO, ch), sE.at[dsti], add=True)
            pltpu.sync_copy(_sl(onesv, ch), sC.at[dsti], add=True)

        def _main(c, carry):
            _chunk(ebase0 + c * _CH, _CH, srcv, dstv)
            return carry

        lax.fori_loop(0, NFULL, _main, 0)
        if TAIL:
            srcv_t, dstv_t = tail_scratch
            _chunk(ebase0 + NFULL * _CH, TAIL, srcv_t, dstv_t)

        plsc.subcore_barrier()

        def _dump_rows(nb, nrows):
            pltpu.sync_copy(sE.at[pl.ds(nb, nrows)],
                            segE_hbm.at[cid, pl.ds(nb, nrows)])
            pltpu.sync_copy(sC.at[pl.ds(nb, nrows)],
                            cnt_hbm.at[cid, pl.ds(nb, nrows)])

        _dump_rows(sid * ROWS, ROWS)
        if REM:
            @pl.when(sid == _NS - 1)
            def _():
                _dump_rows(_NS * ROWS, REM)

    return body(src, dst, P, Q, C, ge, be)


def _sc_agg_call(N, D, E, src, dst, A):
    EPT, NFULL, TAIL = _edge_plan(E)
    ROWS, REM = _row_partition(N)

    mesh = plsc.VectorSubcoreMesh(core_axis_name="c", subcore_axis_name="s")

    scratch = [
        pltpu.VMEM((_CH,), jnp.int32),
        pltpu.VMEM((_CH,), jnp.int32),
        pltpu.VMEM((_CH, D), jnp.float32),
        pltpu.VMEM_SHARED((N, D), jnp.float32),
        pltpu.SemaphoreType.DMA,
    ]
    if TAIL:
        scratch += [pltpu.VMEM((TAIL,), jnp.int32),
                    pltpu.VMEM((TAIL,), jnp.int32)]

    out_type = [jax.ShapeDtypeStruct((_NC, N, D), jnp.float32)]

    @functools.partial(pl.kernel, out_type=out_type, mesh=mesh,
                       scratch_types=scratch)
    def body(src_hbm, dst_hbm, a_hbm, segA_hbm,
             srcv, dstv, bufA, sA, s0, *tail_scratch):
        cid = lax.axis_index("c")
        sid = lax.axis_index("s")
        wid = sid * _NC + cid
        ebase0 = wid * EPT

        z16 = jnp.zeros((_L,), jnp.float32)

        def _zrow(i, carry):
            for j in range(D // _L):
                bufA[i, pl.ds(_L * j, _L)] = z16
            return carry

        lax.fori_loop(0, _CH, _zrow, 0)

        def _zero_rows(base_r, nrows):
            off = 0
            while off < nrows:
                sz = min(_CH, nrows - off)
                zsrc = bufA if sz == _CH else bufA.at[pl.ds(0, sz)]
                pltpu.sync_copy(zsrc, sA.at[pl.ds(base_r + off, sz)])
                off += sz

        _zero_rows(sid * ROWS, ROWS)
        if REM:
            @pl.when(sid == _NS - 1)
            def _():
                _zero_rows(_NS * ROWS, REM)
        plsc.subcore_barrier()

        def _sl(ref, ch):
            return ref if ch == _CH else ref.at[pl.ds(0, ch)]

        def _chunk(ebase, ch, srci, dsti):
            pltpu.sync_copy(src_hbm.at[pl.ds(ebase, ch)], srci)
            pltpu.sync_copy(dst_hbm.at[pl.ds(ebase, ch)], dsti)
            pltpu.async_copy(a_hbm.at[srci], _sl(bufA, ch), s0).wait()
            pltpu.sync_copy(_sl(bufA, ch), sA.at[dsti], add=True)

        def _main(c, carry):
            _chunk(ebase0 + c * _CH, _CH, srcv, dstv)
            return carry

        lax.fori_loop(0, NFULL, _main, 0)
        if TAIL:
            srcv_t, dstv_t = tail_scratch
            _chunk(ebase0 + NFULL * _CH, TAIL, srcv_t, dstv_t)

        plsc.subcore_barrier()

        def _dump_rows(nb, nrows):
            pltpu.sync_copy(sA.at[pl.ds(nb, nrows)],
                            segA_hbm.at[cid, pl.ds(nb, nrows)])

        _dump_rows(sid * ROWS, ROWS)
        if REM:
            @pl.when(sid == _NS - 1)
            def _():
                _dump_rows(_NS * ROWS, REM)

    return body(src, dst, A)[0]


def kernel(x, edge_index, edge_attr, eW, eb, mW, mb, g0, b0, ge, be):
    N, D = x.shape
    E = edge_index.shape[1]
    DE = edge_attr.shape[1]
    f32 = jnp.float32

    eWs, eWe, eWd = eW[:D], eW[D:D + DE], eW[D + DE:]
    mWs, mWe, mWd = mW[:D], mW[D:D + DE], mW[D + DE:]
    src, dst = edge_index[0], edge_index[1]

    BN = 2000

    def _proj(x_ref, ws_ref, wd_ref, ms_ref, md_ref,
              p_ref, q_ref, a_ref, b_ref):
        xb = x_ref[...]
        p_ref[...] = jnp.dot(xb, ws_ref[...], preferred_element_type=f32)
        q_ref[...] = jnp.dot(xb, wd_ref[...], preferred_element_type=f32)
        a_ref[...] = jnp.dot(xb, ms_ref[...], preferred_element_type=f32)
        b_ref[...] = jnp.dot(xb, md_ref[...], preferred_element_type=f32)

    P, Q, A, B = pl.pallas_call(
        _proj,
        grid=(N // BN,),
        in_specs=[pl.BlockSpec((BN, D), lambda i: (i, 0)),
                  pl.BlockSpec((D, DE), lambda i: (0, 0)),
                  pl.BlockSpec((D, DE), lambda i: (0, 0)),
                  pl.BlockSpec((D, D), lambda i: (0, 0)),
                  pl.BlockSpec((D, D), lambda i: (0, 0))],
        out_specs=[pl.BlockSpec((BN, DE), lambda i: (i, 0)),
                   pl.BlockSpec((BN, DE), lambda i: (i, 0)),
                   pl.BlockSpec((BN, D), lambda i: (i, 0)),
                   pl.BlockSpec((BN, D), lambda i: (i, 0))],
        out_shape=[jax.ShapeDtypeStruct((N, DE), f32),
                   jax.ShapeDtypeStruct((N, DE), f32),
                   jax.ShapeDtypeStruct((N, D), f32),
                   jax.ShapeDtypeStruct((N, D), f32)],
    )(x, eWs, eWd, mWs, mWd)

    BE = 16000

    def _cker(ea_ref, we_ref, eb_ref, c_ref):
        ea = ea_ref[...]
        c_ref[...] = ea + jnp.dot(ea, we_ref[...],
                                  preferred_element_type=f32) + eb_ref[...]

    C = pl.pallas_call(
        _cker,
        grid=(E // BE,),
        in_specs=[pl.BlockSpec((BE, DE), lambda i: (i, 0)),
                  pl.BlockSpec((DE, DE), lambda i: (0, 0)),
                  pl.BlockSpec((1, DE), lambda i: (0, 0))],
        out_specs=pl.BlockSpec((BE, DE), lambda i: (i, 0)),
        out_shape=jax.ShapeDtypeStruct((E, DE), f32),
    )(edge_attr, eWe, eb.reshape(1, DE))

    eo, pE, pC = _sc_edge_call(N, D, DE, E, src, dst, P, Q, C, ge, be)
    pA = _sc_agg_call(N, D, E, src, dst, A)

    BN2 = 2000

    def _fin(x_ref, pa_ref, pe_ref, pc_ref, b_ref, mwe_ref, mb_ref,
             g0_ref, b0_ref, o_ref):
        segA = pa_ref[0] + pa_ref[1]
        segE = pe_ref[0] + pe_ref[1]
        cnt = (pc_ref[0] + pc_ref[1])[:, :1]
        sums = segA + jnp.dot(segE, mwe_ref[...], preferred_element_type=f32)
        sums = sums + cnt * (b_ref[...] + mb_ref[...])
        dh = sums / jnp.maximum(cnt, 1.0)
        h = x_ref[...] + dh
        mu = jnp.mean(h, axis=-1, keepdims=True)
        var = jnp.mean((h - mu) ** 2, axis=-1, keepdims=True)
        o_ref[...] = (h - mu) / jnp.sqrt(var + 1e-5) * g0_ref[...] + b0_ref[...]

    x_out = pl.pallas_call(
        _fin,
        grid=(N // BN2,),
        in_specs=[pl.BlockSpec((BN2, D), lambda i: (i, 0)),
                  pl.BlockSpec((_NC, BN2, D), lambda i: (0, i, 0)),
                  pl.BlockSpec((_NC, BN2, DE), lambda i: (0, i, 0)),
                  pl.BlockSpec((_NC, BN2, DE), lambda i: (0, i, 0)),
                  pl.BlockSpec((BN2, D), lambda i: (i, 0)),
                  pl.BlockSpec((DE, D), lambda i: (0, 0)),
                  pl.BlockSpec((1, D), lambda i: (0, 0)),
                  pl.BlockSpec((1, D), lambda i: (0, 0)),
                  pl.BlockSpec((1, D), lambda i: (0, 0))],
        out_specs=pl.BlockSpec((BN2, D), lambda i: (i, 0)),
        out_shape=jax.ShapeDtypeStruct((N, D), f32),
    )(x, pA, pE, pC, B, mWe, mb.reshape(1, D), g0.reshape(1, D),
      b0.reshape(1, D))

    return (x_out, eo)
